# baseline (speedup 1.0000x reference)
import functools

import jax
import jax.numpy as jnp
from jax import lax
from jax.experimental import pallas as pl
from jax.experimental.pallas import tpu as pltpu
from jax.experimental.pallas import tpu_sc as plsc

B = 64
S = 1025
SP = 1152
D = 768
K = 8
NC = 2
NS = 16
L = 16
TCH = 128
BCH = 32
NSTR = 4
HCH = TCH // NSTR


def _dist_body(*refs):
    ys = refs[:NSTR]
    g_ref = refs[NSTR]
    out_ref = refs[NSTR + 1]
    c = pl.program_id(0)
    g = g_ref[...]
    gn = jnp.sqrt(jnp.sum(g * g, axis=2))
    parts = []
    for h, y_ref in enumerate(ys):
        rows = y_ref[...]
        num = jnp.sum(rows * g, axis=2)
        ln = jnp.sqrt(jnp.sum(rows * rows, axis=2))
        dist = num / jnp.maximum(gn * ln, 1e-8)
        rid = (c * TCH + h * HCH
               + lax.broadcasted_iota(jnp.int32, (HCH, BCH), 0))
        valid = (rid > 0) & (rid < S)
        parts.append(jnp.where(valid, dist, -jnp.inf))
    dist = jnp.concatenate(parts, axis=0)
    out_ref[...] = jnp.swapaxes(dist, 0, 1)


_MAXBLK = (S - 1) // HCH


def _distances(y):
    return pl.pallas_call(
        _dist_body,
        grid=(SP // TCH, B // BCH),
        in_specs=[
            pl.BlockSpec((HCH, BCH, D),
                         lambda c, bc, h=h: (jnp.minimum(NSTR * c + h, _MAXBLK),
                                             bc, 0))
            for h in range(NSTR)
        ] + [
            pl.BlockSpec((1, BCH, D), lambda c, bc: (0, bc, 0)),
        ],
        out_specs=pl.BlockSpec((BCH, TCH), lambda c, bc: (bc, c)),
        out_shape=jax.ShapeDtypeStruct((B, SP), jnp.float32),
    )(*([y] * (NSTR + 1)))


@functools.lru_cache(maxsize=None)
def _topk_crop_kernel():
    @functools.partial(
        pl.kernel,
        out_type=jax.ShapeDtypeStruct((B, K, D), jnp.float32),
        mesh=plsc.VectorSubcoreMesh(core_axis_name="c", subcore_axis_name="s"),
        scratch_types=[
            pltpu.VMEM((SP,), jnp.float32),
            pltpu.VMEM((L,), jnp.int32),
            pltpu.VMEM((L, D), jnp.float32),
            pltpu.SemaphoreType.DMA,
        ],
        compiler_params=pltpu.CompilerParams(needs_layout_passes=False),
    )
    def _topk_crop(dist_hbm, y2d_hbm, out_hbm, dist_v, idx_v, rows_v, sem):
        wid = lax.axis_index("s") * NC + lax.axis_index("c")
        iota = jnp.arange(L, dtype=jnp.int32)
        for i in range(B // (NC * NS)):
            b = wid * (B // (NC * NS)) + i
            pltpu.sync_copy(dist_hbm.at[b], dist_v)
            tv, ti = plsc.sort_key_val(dist_v[pl.ds(0, L)], iota,
                                       descending=True)
            for j in range(1, SP // L):
                sv, si = plsc.sort_key_val(dist_v[pl.ds(j * L, L)],
                                           iota + (j * L), descending=True)
                rv = lax.rev(sv, (0,))
                ri = lax.rev(si, (0,))
                m = tv >= rv
                hi = jnp.where(m, tv, rv)
                hx = jnp.where(m, ti, ri)
                tv, ti = plsc.sort_key_val(hi, hx, descending=True)
            idx_v[...] = ti * B + b
            pltpu.async_copy(y2d_hbm.at[idx_v], rows_v, sem).wait()
            pltpu.sync_copy(rows_v.at[pl.ds(0, K)], out_hbm.at[b])

    return _topk_crop


def kernel(x, images):
    del images
    y = jnp.transpose(x, (1, 0, 2))
    dist = _distances(y)
    y2d = y.reshape(S * B, D)
    return _topk_crop_kernel()(dist, y2d)


# speedup vs baseline: 1.0246x; 1.0246x over previous
import functools

import jax
import jax.numpy as jnp
from jax import lax
from jax.experimental import pallas as pl
from jax.experimental.pallas import tpu as pltpu
from jax.experimental.pallas import tpu_sc as plsc

B = 64
S = 1025
SP = 1152
D = 768
K = 8
NC = 2
NS = 16
L = 16
TCH = 128
BCH = 32
NSTR = 2
HCH = TCH // NSTR


def _dist_body(*refs):
    ys = refs[:NSTR]
    g_ref = refs[NSTR]
    out_ref = refs[NSTR + 1]
    c = pl.program_id(0)
    g = g_ref[...]
    gn = jnp.sqrt(jnp.sum(g * g, axis=2))
    parts = []
    for h, y_ref in enumerate(ys):
        rows = y_ref[...]
        num = jnp.sum(rows * g, axis=2)
        ln = jnp.sqrt(jnp.sum(rows * rows, axis=2))
        dist = num / jnp.maximum(gn * ln, 1e-8)
        rid = (c * TCH + h * HCH
               + lax.broadcasted_iota(jnp.int32, (HCH, BCH), 0))
        valid = (rid > 0) & (rid < S)
        parts.append(jnp.where(valid, dist, -jnp.inf))
    dist = jnp.concatenate(parts, axis=0)
    out_ref[...] = jnp.swapaxes(dist, 0, 1)


_MAXBLK = (S - 1) // HCH


def _distances(y):
    return pl.pallas_call(
        _dist_body,
        grid=(SP // TCH, B // BCH),
        in_specs=[
            pl.BlockSpec((HCH, BCH, D),
                         lambda c, bc, h=h: (jnp.minimum(NSTR * c + h, _MAXBLK),
                                             bc, 0))
            for h in range(NSTR)
        ] + [
            pl.BlockSpec((1, BCH, D), lambda c, bc: (0, bc, 0)),
        ],
        out_specs=pl.BlockSpec((BCH, TCH), lambda c, bc: (bc, c)),
        out_shape=jax.ShapeDtypeStruct((B, SP), jnp.float32),
    )(*([y] * (NSTR + 1)))


@functools.lru_cache(maxsize=None)
def _topk_crop_kernel():
    @functools.partial(
        pl.kernel,
        out_type=jax.ShapeDtypeStruct((B, K, D), jnp.float32),
        mesh=plsc.VectorSubcoreMesh(core_axis_name="c", subcore_axis_name="s"),
        scratch_types=[
            pltpu.VMEM((SP,), jnp.float32),
            pltpu.VMEM((L,), jnp.int32),
            pltpu.VMEM((L, D), jnp.float32),
            pltpu.SemaphoreType.DMA,
        ],
        compiler_params=pltpu.CompilerParams(needs_layout_passes=False),
    )
    def _topk_crop(dist_hbm, y2d_hbm, out_hbm, dist_v, idx_v, rows_v, sem):
        wid = lax.axis_index("s") * NC + lax.axis_index("c")
        iota = jnp.arange(L, dtype=jnp.int32)
        for i in range(B // (NC * NS)):
            b = wid * (B // (NC * NS)) + i
            pltpu.sync_copy(dist_hbm.at[b], dist_v)
            tv, ti = plsc.sort_key_val(dist_v[pl.ds(0, L)], iota,
                                       descending=True)
            for j in range(1, SP // L):
                sv, si = plsc.sort_key_val(dist_v[pl.ds(j * L, L)],
                                           iota + (j * L), descending=True)
                rv = lax.rev(sv, (0,))
                ri = lax.rev(si, (0,))
                m = tv >= rv
                hi = jnp.where(m, tv, rv)
                hx = jnp.where(m, ti, ri)
                tv, ti = plsc.sort_key_val(hi, hx, descending=True)
            idx_v[...] = ti * B + b
            pltpu.async_copy(y2d_hbm.at[idx_v], rows_v, sem).wait()
            pltpu.sync_copy(rows_v.at[pl.ds(0, K)], out_hbm.at[b])

    return _topk_crop


def kernel(x, images):
    del images
    y = jnp.transpose(x, (1, 0, 2))
    dist = _distances(y)
    y2d = y.reshape(S * B, D)
    return _topk_crop_kernel()(dist, y2d)
